# Initial kernel scaffold; baseline (speedup 1.0000x reference)
import jax
import jax.numpy as jnp
from jax.experimental import pallas as pl


def kernel(z, embed_weight):
    raise NotImplementedError("write your pallas kernel here")


# speedup vs baseline: 1.2910x; 1.2910x over previous
import functools

import jax
import jax.numpy as jnp
from jax import lax
from jax.experimental import pallas as pl
from jax.experimental.pallas import tpu as pltpu
from jax.experimental.pallas import tpu_sc as plsc

_NUM_EMBED = 8192
_EMBED_DIM = 256
_BETA = 0.25
_TOKENS = 16384
_TBLK = 512
_NTB = _TOKENS // _TBLK
_CBLK = 1024
_NCB = _NUM_EMBED // _CBLK


def _normalize_cb_body(w_ref, n_ref, out_ref):
    out_ref[...] = w_ref[...] / n_ref[...]


def _dist_argmax_body(z_ref, n_ref, ncb_ref, idx_ref):
    zb = z_ref[...]
    nz = zb / n_ref[...]
    dist = lax.dot_general(
        nz, ncb_ref[...], (((1,), (1,)), ((), ())),
        preferred_element_type=jnp.float32)
    m = jnp.max(dist, axis=1, keepdims=True)
    ii = lax.broadcasted_iota(jnp.int32, dist.shape, 1)
    first = jnp.min(jnp.where(dist == m, ii, jnp.int32(_NUM_EMBED)), axis=1)
    idx_ref[0, 0, :] = first.astype(jnp.int32)


def _finalize_body(zq_ref, z_ref, st_ref, out_ref):
    t = pl.program_id(0)

    @pl.when(t == 0)
    def _init():
        out_ref[...] = jnp.zeros((1, 1), jnp.float32)

    zb = z_ref[...]
    d = zq_ref[...] - zb
    st_ref[...] = zb + d
    out_ref[...] += jnp.sum(d * d).reshape(1, 1)

    @pl.when(t == pl.num_programs(0) - 1)
    def _fin():
        out_ref[...] = out_ref[...] * ((1.0 + _BETA) / (_TOKENS * _EMBED_DIM))


@functools.cache
def _make_sc_gather():
    info = plsc.get_sparse_core_info()
    nc, ns = info.num_cores, info.num_subcores
    nw = nc * ns
    b_per_w = _TOKENS // nw
    ch = 128
    nch = b_per_w // ch
    mesh = plsc.VectorSubcoreMesh(core_axis_name="c", subcore_axis_name="s")

    @functools.partial(
        pl.kernel, mesh=mesh,
        out_type=jax.ShapeDtypeStruct((_TOKENS, _EMBED_DIM), jnp.float32),
        scratch_types=[
            pltpu.VMEM((ch,), jnp.int32),
            pltpu.VMEM((ch, _EMBED_DIM), jnp.float32),
            pltpu.SemaphoreType.DMA,
        ],
    )
    def gather(table_hbm, idx_hbm, out_hbm, idx_v, rows_v, sem):
        wid = lax.axis_index("s") * nc + lax.axis_index("c")
        base = wid * b_per_w

        def body(c, carry):
            off = base + c * ch
            pltpu.sync_copy(idx_hbm.at[pl.ds(off, ch)], idx_v)
            pltpu.async_copy(table_hbm.at[idx_v], rows_v, sem).wait()
            pltpu.sync_copy(rows_v, out_hbm.at[pl.ds(off, ch)])
            return carry

        lax.fori_loop(0, nch, body, 0)

    return gather


def kernel(z, embed_weight):
    b, c, h, w = z.shape
    zp = jnp.transpose(z, (0, 2, 3, 1)).reshape(_TOKENS, _EMBED_DIM)

    z_ss = jnp.sum(z * z, axis=1)
    n_z = jnp.maximum(jnp.sqrt(z_ss), 1e-12).reshape(_TOKENS, 1)
    cb_ss = jnp.sum(embed_weight * embed_weight, axis=1, keepdims=True)
    n_c = jnp.maximum(jnp.sqrt(cb_ss), 1e-12)
    ncb = pl.pallas_call(
        _normalize_cb_body,
        grid=(_NCB,),
        in_specs=[
            pl.BlockSpec((_CBLK, _EMBED_DIM), lambda i: (i, 0)),
            pl.BlockSpec((_CBLK, 1), lambda i: (i, 0)),
        ],
        out_specs=pl.BlockSpec((_CBLK, _EMBED_DIM), lambda i: (i, 0)),
        out_shape=jax.ShapeDtypeStruct((_NUM_EMBED, _EMBED_DIM), jnp.float32),
    )(embed_weight, n_c)

    idx3 = pl.pallas_call(
        _dist_argmax_body,
        grid=(_NTB,),
        in_specs=[
            pl.BlockSpec((_TBLK, _EMBED_DIM), lambda i: (i, 0)),
            pl.BlockSpec((_TBLK, 1), lambda i: (i, 0)),
            pl.BlockSpec((_NUM_EMBED, _EMBED_DIM), lambda i: (0, 0)),
        ],
        out_specs=pl.BlockSpec((1, 1, _TBLK), lambda i: (i, 0, 0)),
        out_shape=jax.ShapeDtypeStruct((_NTB, 1, _TBLK), jnp.int32),
    )(zp, n_z, ncb)
    encoding_indices = idx3.reshape(_TOKENS)

    zq_flat = _make_sc_gather()(embed_weight, encoding_indices)

    zq_st, loss2 = pl.pallas_call(
        _finalize_body,
        grid=(_NTB,),
        in_specs=[
            pl.BlockSpec((_TBLK, _EMBED_DIM), lambda i: (i, 0)),
            pl.BlockSpec((_TBLK, _EMBED_DIM), lambda i: (i, 0)),
        ],
        out_specs=[
            pl.BlockSpec((_TBLK, _EMBED_DIM), lambda i: (i, 0)),
            pl.BlockSpec((1, 1), lambda i: (0, 0)),
        ],
        out_shape=[
            jax.ShapeDtypeStruct((_TOKENS, _EMBED_DIM), jnp.float32),
            jax.ShapeDtypeStruct((1, 1), jnp.float32),
        ],
    )(zq_flat, zp)
    loss = loss2[0, 0]

    z_q = jnp.transpose(zq_st.reshape(b, h, w, c), (0, 3, 1, 2))
    return z_q, loss, encoding_indices
